# baseline (speedup 1.0000x reference)
import functools

import numpy as np
import jax
import jax.numpy as jnp
from jax import lax
from jax.experimental import pallas as pl
from jax.experimental.pallas import tpu as pltpu
from jax.experimental.pallas import tpu_sc as plsc

_N = 4096
_K = 32
_CL = 64
_D = 448
_DP = 512
_R = 0.1
_G = 32768
_PAD = 1058
_OFFS = (0, 1, 32, 33, 1024, 1025, 1056, 1057)
_RB = 256
_AB = 128
_HB = 512
_HI = 1e30

_INTERPRET = False


def _posenc_sel():
  s96 = np.zeros((1, 96), np.float32)
  for c in range(8):
    sh = ((c >> 2) & 1, (c >> 1) & 1, c & 1)
    for d in range(3):
      for a in range(4):
        s96[0, c * 12 + d * 4 + a] = float(sh[d])
  ang = np.zeros((1, 96), np.float32)
  for col in range(96):
    ang[0, col] = float(col % 4 + 1)
  return jnp.asarray(s96), jnp.asarray(ang)


def _bf(x):
  return x.astype(jnp.bfloat16)


def _hist_body(pos_ref, h_ref):
  b = pl.program_id(0)
  p = pos_ref[...]
  vx = jnp.clip(jnp.floor(p[:, 0:1] * 32.0), 0.0, 31.0)
  vy = jnp.clip(jnp.floor(p[:, 1:2] * 32.0), 0.0, 31.0)
  vz = jnp.clip(jnp.floor(p[:, 2:3] * 32.0), 0.0, 31.0)
  flat = (vx * 32.0 + vy) * 32.0 + vz
  col = (lax.broadcasted_iota(jnp.int32, (_N, _HB), 1)
         + b * _HB).astype(jnp.float32)
  e = (flat == col).astype(jnp.float32)
  h_ref[...] = jnp.sum(e, axis=0).reshape(1, 1, _HB)


def _hist(pos):
  nb = _G // _HB
  out = pl.pallas_call(
      _hist_body,
      grid=(nb,),
      in_specs=[pl.BlockSpec((_N, 3), lambda i: (0, 0))],
      out_specs=pl.BlockSpec((1, 1, _HB), lambda i: (i, 0, 0)),
      out_shape=jax.ShapeDtypeStruct((nb, 1, _HB), jnp.float32),
      interpret=_INTERPRET,
  )(pos)
  return out.reshape(_G)


def _prep_body(enc_ref, d8_ref, latcat_ref, wstack_ref, uv_ref):
  wb = _bf(wstack_ref[...])
  u = lax.dot_general(_bf(latcat_ref[...]), wb, (((1,), (0,)), ((), ())),
                      preferred_element_type=jnp.float32)
  d8 = d8_ref[...]
  blk = d8.shape[0]
  d8r = jnp.broadcast_to(d8.reshape(blk, 8, 1), (blk, 8, _CL)).reshape(
      blk, 8 * _CL)
  enct = jnp.concatenate([enc_ref[...]] * 8, axis=1)
  corr = d8r * enct
  v = lax.dot_general(_bf(corr), wb, (((1,), (0,)), ((), ())),
                      preferred_element_type=jnp.float32)
  uv_ref[...] = jnp.concatenate([u, v], axis=1)


def _prep(enc_row, d8, latcat, wstack):
  nb = 16
  blk = _G // nb
  return pl.pallas_call(
      _prep_body,
      grid=(nb,),
      in_specs=[
          pl.BlockSpec((1, _CL), lambda i: (0, 0)),
          pl.BlockSpec((blk, 8), lambda i: (i, 0)),
          pl.BlockSpec((blk, 8 * _CL), lambda i: (i, 0)),
          pl.BlockSpec((8 * _CL, _CL), lambda i: (0, 0)),
      ],
      out_specs=pl.BlockSpec((blk, 2 * _CL), lambda i: (i, 0)),
      out_shape=jax.ShapeDtypeStruct((_G, 2 * _CL), jnp.float32),
      interpret=_INTERPRET,
  )(enc_row, d8, latcat, wstack)


def _sc_gather(uv, sx, sy, sz):
  info = plsc.get_sparse_core_info()
  nc, ns = info.num_cores, info.num_subcores
  nw = nc * ns
  bpw = _N // nw

  mesh = plsc.VectorSubcoreMesh(core_axis_name="c", subcore_axis_name="s")

  @functools.partial(
      pl.kernel,
      mesh=mesh,
      out_type=jax.ShapeDtypeStruct((_N, 2 * _CL), jnp.float32),
      scratch_types=[
          pltpu.VMEM((bpw,), jnp.float32),
          pltpu.VMEM((bpw,), jnp.float32),
          pltpu.VMEM((bpw,), jnp.float32),
          pltpu.VMEM((bpw,), jnp.int32),
          pltpu.VMEM((bpw, 2 * _CL), jnp.float32),
          pltpu.SemaphoreType.DMA,
      ],
  )
  def k(uv_hbm, sx_hbm, sy_hbm, sz_hbm, out_hbm, xv, yv, zv, idxv, rows, sem):
    wid = lax.axis_index("s") * nc + lax.axis_index("c")
    base = wid * bpw
    pltpu.sync_copy(sx_hbm.at[pl.ds(base, bpw)], xv)
    pltpu.sync_copy(sy_hbm.at[pl.ds(base, bpw)], yv)
    pltpu.sync_copy(sz_hbm.at[pl.ds(base, bpw)], zv)
    for kk in range(bpw // 16):
      s = pl.ds(kk * 16, 16)
      ix = xv[s].astype(jnp.int32)
      iy = yv[s].astype(jnp.int32)
      iz = zv[s].astype(jnp.int32)
      idxv[s] = (ix * 1024 + iy * 32) + iz
    pltpu.async_copy(uv_hbm.at[idxv], rows, sem).wait()
    pltpu.sync_copy(rows, out_hbm.at[pl.ds(base, bpw)])

  return k(uv, sx, sy, sz)


def _asm_body(repr_ref, uvg_ref, pos_ref, box_ref, s96_ref, ang_ref, wpe_ref,
              a_ref):
  pos = pos_ref[...]
  box = box_ref[...]
  uni = pos / box
  s31 = 31.0 * uni
  idx3 = jnp.floor(s31)
  lenr = box / 31.0
  pl3 = pos / lenr
  d3 = idx3 - pl3

  def widen(m):
    cols = [m[:, 0:1]] * 4 + [m[:, 1:2]] * 4 + [m[:, 2:3]] * 4
    m12 = jnp.concatenate(cols, axis=1)
    return jnp.concatenate([m12] * 8, axis=1)

  d96 = widen(d3) + s96_ref[...]
  shifted = (d96 + 1.0) * 0.5
  m = shifted * ang_ref[...]
  pe = lax.dot_general(_bf(jnp.cos(m)), _bf(wpe_ref[...]),
                       (((1,), (0,)), ((), ())),
                       preferred_element_type=jnp.float32)
  uvg = uvg_ref[...]
  alpha = uvg[:, :_CL] + pe
  calpha = uvg[:, _CL:] + pe
  a_ref[...] = jnp.concatenate(
      [repr_ref[...], alpha, calpha, calpha - alpha], axis=1)


def _asm(arep, uvg, pos, box_row, s96, ang, wpe):
  return pl.pallas_call(
      _asm_body,
      out_shape=jax.ShapeDtypeStruct((_N, _D), jnp.float32),
      interpret=_INTERPRET,
  )(arep, uvg, pos, box_row, s96, ang, wpe)


def _sel_body(pb_ref, pt_ref, idx_ref, d2_ref):
  pt = pt_ref[...]
  pb = pb_ref[...]
  dx = pb[:, 0:1] - pt[0:1, :]
  dy = pb[:, 1:2] - pt[1:2, :]
  dz = pb[:, 2:3] - pt[2:3, :]
  d2 = (dx * dx + dy * dy) + dz * dz
  dist = jnp.sqrt(d2 + 1e-16)
  hi = jnp.float32(_HI)
  d2m = jnp.where(dist < _R, d2, hi)
  iota = lax.broadcasted_iota(jnp.int32, (_RB, _N), 1)
  big_i = jnp.int32(_N)
  for k in range(_K):
    m = jnp.min(d2m, axis=1, keepdims=True)
    colc = jnp.where(d2m == m, iota, big_i)
    col = jnp.min(colc, axis=1, keepdims=True)
    d2m = jnp.where(iota == col, hi, d2m)
    idx_ref[:, k:k + 1] = jnp.minimum(col, _N - 1)
    d2_ref[:, k:k + 1] = m


def _sel(pos, pos_t):
  nb = _N // _RB
  return pl.pallas_call(
      _sel_body,
      grid=(nb,),
      in_specs=[
          pl.BlockSpec((_RB, 3), lambda i: (i, 0)),
          pl.BlockSpec((3, _N), lambda i: (0, 0)),
      ],
      out_specs=[
          pl.BlockSpec((_RB, _K), lambda i: (i, 0)),
          pl.BlockSpec((_RB, _K), lambda i: (i, 0)),
      ],
      out_shape=[
          jax.ShapeDtypeStruct((_N, _K), jnp.int32),
          jax.ShapeDtypeStruct((_N, _K), jnp.float32),
      ],
      interpret=_INTERPRET,
  )(pos, pos_t)


def _sc_gather_rows(a, idxflat):
  info = plsc.get_sparse_core_info()
  nc, ns = info.num_cores, info.num_subcores
  nw = nc * ns
  total = _N * _K
  bpw = total // nw
  chunk = 128
  nch = bpw // chunk

  mesh = plsc.VectorSubcoreMesh(core_axis_name="c", subcore_axis_name="s")

  @functools.partial(
      pl.kernel,
      mesh=mesh,
      out_type=jax.ShapeDtypeStruct((total, _DP), jnp.float32),
      scratch_types=[
          pltpu.VMEM((chunk,), jnp.int32),
          pltpu.VMEM((chunk, _DP), jnp.float32),
          pltpu.SemaphoreType.DMA,
      ],
  )
  def k(a_hbm, idx_hbm, out_hbm, idxv, rows, sem):
    wid = lax.axis_index("s") * nc + lax.axis_index("c")
    base = wid * bpw

    def body(j, carry):
      off = base + j * chunk
      pltpu.sync_copy(idx_hbm.at[pl.ds(off, chunk)], idxv)
      pltpu.async_copy(a_hbm.at[idxv], rows, sem).wait()
      pltpu.sync_copy(rows, out_hbm.at[pl.ds(off, chunk)])
      return carry

    lax.fori_loop(0, nch, body, 0)

  return k(a, idxflat)


def _attn_body(ab_ref, ag_ref, d2v_ref, o_ref):
  d2v = d2v_ref[...]
  dist = jnp.sqrt(d2v + 1e-16)
  valid = d2v < 1e29
  mask = (dist < _R).astype(jnp.float32)
  distp = dist + (dist < 1e-6).astype(jnp.float32) * 1e9
  inv = 1.0 / (distp * distp)
  ab16 = _bf(ab_ref[...]).astype(jnp.float32)
  lcols = []
  for k in range(_K):
    mem_k = ag_ref[k] * inv[:, k:k + 1]
    prod = ab16 * _bf(mem_k).astype(jnp.float32)
    lcols.append(jnp.sum(prod, axis=1, keepdims=True))
  logits = jnp.concatenate(lcols, axis=1) / jnp.sqrt(jnp.float32(448.0))
  logits = logits + 1e9 * (mask - 1.0)
  logits = jnp.where(valid, logits, jnp.float32(-_HI))
  mx = jnp.max(logits, axis=1, keepdims=True)
  e = jnp.exp(logits - mx)
  attn = e / jnp.sum(e, axis=1, keepdims=True)
  acc = jnp.zeros((_AB, _DP), jnp.float32)
  for k in range(_K):
    mem_k = ag_ref[k] * inv[:, k:k + 1]
    acc = acc + attn[:, k:k + 1] * mem_k
  o_ref[...] = acc


def _attn(a, ag3, d2v):
  nb = _N // _AB
  return pl.pallas_call(
      _attn_body,
      grid=(nb,),
      in_specs=[
          pl.BlockSpec((_AB, _DP), lambda i: (i, 0)),
          pl.BlockSpec((_K, _AB, _DP), lambda i: (0, i, 0)),
          pl.BlockSpec((_AB, _K), lambda i: (i, 0)),
      ],
      out_specs=pl.BlockSpec((_AB, _DP), lambda i: (i, 0)),
      out_shape=jax.ShapeDtypeStruct((_N, _DP), jnp.float32),
      interpret=_INTERPRET,
  )(a, ag3, d2v)


def kernel(enc_params, atom_representation, latent_encoding, atom_positions,
           box_size, proj_weights):
  pos = atom_positions
  hist = _hist(pos)
  hpad = jnp.pad(hist, (0, _PAD))
  d8 = jnp.stack([hpad[o:o + _G] for o in _OFFS], axis=1)
  latpad = jnp.pad(latent_encoding.reshape(_G, _CL), ((0, _PAD), (0, 0)))
  latcat = jnp.concatenate([latpad[o:o + _G] for o in _OFFS], axis=1)
  wstack = jnp.concatenate(
      [proj_weights[76 * c:76 * c + 64] for c in range(8)], axis=0)
  uv = _prep(enc_params.reshape(1, _CL), d8, latcat, wstack)

  s31 = (pos / box_size[None, :]) * 31.0
  uvg = _sc_gather(uv, s31[:, 0], s31[:, 1], s31[:, 2])

  wpe = jnp.concatenate(
      [proj_weights[76 * c + 64:76 * c + 76] for c in range(8)], axis=0)
  s96, ang = _posenc_sel()
  a = _asm(atom_representation, uvg, pos, box_size.reshape(1, 3), s96, ang,
           wpe)

  idx, d2v = _sel(pos, pos.T)
  a_pad = jnp.pad(a, ((0, 0), (0, _DP - _D)))
  idx_kmaj = idx.T.reshape(-1)
  ag = _sc_gather_rows(a_pad, idx_kmaj)
  ag3 = ag.reshape(_K, _N, _DP)
  return _attn(a_pad, ag3, d2v)[:, :_D]
